# baseline (speedup 1.0000x reference)
import jax
import jax.numpy as jnp
from jax import lax
from jax.experimental import pallas as pl
from jax.experimental.pallas import tpu as pltpu
from jax.experimental.pallas import tpu_sc as plsc

B, S, H = 4, 2048, 768
NC, NS = 2, 16
NW = NC * NS
PPW = S // NW
CP = 16
NCHUNK = PPW // CP
LANES = 16
KV = H // LANES
EPS = 1e-12


def _rsqrt(x):
    xi = lax.bitcast_convert_type(x, jnp.int32)
    yi = jnp.int32(0x5F3759DF) - (xi >> 1)
    y = lax.bitcast_convert_type(yi, jnp.float32)
    for _ in range(3):
        y = y * (1.5 - 0.5 * x * y * y)
    return y


def _body(ids_hbm, tids_hbm, word_hbm, pos_hbm, type_hbm, gamma_hbm, beta_hbm,
          out_hbm, idsbuf, tidbuf, wordbuf, posbuf, tbuf, dbuf, gbuf, bbuf, sem):
    wid = lax.axis_index("s") * NC + lax.axis_index("c")
    pltpu.sync_copy(type_hbm, tbuf)
    pltpu.sync_copy(gamma_hbm, gbuf)
    pltpu.sync_copy(beta_hbm, bbuf)

    def dinit(k, _):
        off = pl.ds(k * LANES, LANES)
        dbuf[off] = tbuf[1, off] - tbuf[0, off]
        return 0
    lax.fori_loop(0, KV, dinit, 0)

    for c in range(NCHUNK):
        p0 = wid * PPW + c * CP
        for b in range(B):
            pltpu.sync_copy(ids_hbm.at[b, pl.ds(p0, CP)],
                            idsbuf.at[pl.ds(b * CP, CP)])
            pltpu.sync_copy(tids_hbm.at[b, pl.ds(p0, CP)],
                            tidbuf.at[pl.ds(b * CP, CP)])
        gather = pltpu.async_copy(word_hbm.at[idsbuf], wordbuf, sem)
        pltpu.sync_copy(pos_hbm.at[pl.ds(p0, CP)], posbuf)
        gather.wait()

        lane = lax.iota(jnp.int32, LANES)
        for b in range(B):
            tfv = tidbuf[pl.ds(b * CP, CP)].astype(jnp.float32)

            def tok_body(i, _, b=b, tfv=tfv):
                r = b * CP + i
                tid_f = jnp.sum(jnp.where(lane == i, tfv, 0.0))

                def k1(k, carry):
                    s, q = carry
                    off = pl.ds(k * LANES, LANES)
                    v = (wordbuf[r, off] + posbuf[i, off]
                         + tbuf[0, off] + tid_f * dbuf[off])
                    wordbuf[r, off] = v
                    return s + v, q + v * v
                z = jnp.zeros((LANES,), jnp.float32)
                s, q = lax.fori_loop(0, KV, k1, (z, z))
                mean = jnp.sum(s) * (1.0 / H)
                var = jnp.maximum(jnp.sum(q) * (1.0 / H) - mean * mean, 0.0)
                inv = _rsqrt(var + EPS)
                shift = mean * inv

                def k2(k, _):
                    off = pl.ds(k * LANES, LANES)
                    v = wordbuf[r, off]
                    wordbuf[r, off] = (v * inv - shift) * gbuf[off] + bbuf[off]
                    return 0
                lax.fori_loop(0, KV, k2, 0)
                return 0
            lax.fori_loop(0, CP, tok_body, 0)

        for b in range(B):
            pltpu.sync_copy(wordbuf.at[pl.ds(b * CP, CP)],
                            out_hbm.at[b, pl.ds(p0, CP)])


def kernel(input_ids, token_type_ids, word_emb, pos_emb, type_emb, gamma, beta):
    mesh = plsc.VectorSubcoreMesh(core_axis_name="c", subcore_axis_name="s",
                                  num_cores=NC, num_subcores=NS)
    k = pl.kernel(
        _body,
        out_type=jax.ShapeDtypeStruct((B, S, H), jnp.float32),
        mesh=mesh,
        compiler_params=pltpu.CompilerParams(needs_layout_passes=False),
        scratch_types=[
            pltpu.VMEM((B * CP,), jnp.int32),
            pltpu.VMEM((B * CP,), jnp.int32),
            pltpu.VMEM((B * CP, H), jnp.float32),
            pltpu.VMEM((CP, H), jnp.float32),
            pltpu.VMEM((2, H), jnp.float32),
            pltpu.VMEM((H,), jnp.float32),
            pltpu.VMEM((H,), jnp.float32),
            pltpu.VMEM((H,), jnp.float32),
            pltpu.SemaphoreType.DMA,
        ],
    )
    return k(input_ids, token_type_ids, word_emb, pos_emb, type_emb,
             gamma, beta)


# speedup vs baseline: 1.5511x; 1.5511x over previous
import jax
import jax.numpy as jnp
from jax import lax
from jax.experimental import pallas as pl
from jax.experimental.pallas import tpu as pltpu
from jax.experimental.pallas import tpu_sc as plsc

B, S, H = 4, 2048, 768
NC, NS = 2, 16
NW = NC * NS
PPW = S // NW
CP = 16
TPC = B * CP
NCHUNK = PPW // CP
LANES = 16
KV = H // LANES
EPS = 1e-12


def _rsqrt(x):
    xi = lax.bitcast_convert_type(x, jnp.int32)
    yi = jnp.int32(0x5F3759DF) - (xi >> 1)
    y = lax.bitcast_convert_type(yi, jnp.float32)
    for _ in range(3):
        y = y * (1.5 - 0.5 * x * y * y)
    return y


def _body(ids_hbm, tids_hbm, word_hbm, pos_hbm, type_hbm, gamma_hbm, beta_hbm,
          out_hbm, idsbuf, tidbuf, wordbuf0, wordbuf1, posbuf, tbuf, dbuf,
          gbuf, bbuf, semg0, semg1, semo0, semo1):
    wid = lax.axis_index("s") * NC + lax.axis_index("c")
    wordbufs = (wordbuf0, wordbuf1)
    semgs = (semg0, semg1)
    semos = (semo0, semo1)
    pltpu.sync_copy(type_hbm, tbuf)
    pltpu.sync_copy(gamma_hbm, gbuf)
    pltpu.sync_copy(beta_hbm, bbuf)
    for k in range(KV):
        kk = slice(k * LANES, (k + 1) * LANES)
        dbuf[kk] = tbuf[1, kk] - tbuf[0, kk]

    def stage_ids(c, slot):
        p0 = wid * PPW + c * CP
        for b in range(B):
            pltpu.sync_copy(ids_hbm.at[b, pl.ds(p0, CP)],
                            idsbuf.at[slot, pl.ds(b * CP, CP)])
            pltpu.sync_copy(tids_hbm.at[b, pl.ds(p0, CP)],
                            tidbuf.at[slot, pl.ds(b * CP, CP)])

    def start_gather(c, slot):
        return pltpu.async_copy(word_hbm.at[idsbuf.at[slot]],
                                wordbufs[slot], semgs[slot])

    lane = lax.iota(jnp.int32, LANES)
    out_dmas = [None, None]

    stage_ids(0, 0)
    gathers = [start_gather(0, 0), None]
    for c in range(NCHUNK):
        buf = c & 1
        nbuf = 1 - buf
        if c + 1 < NCHUNK:
            if out_dmas[nbuf] is not None:
                for dma in out_dmas[nbuf]:
                    dma.wait()
            stage_ids(c + 1, nbuf)
            gathers[nbuf] = start_gather(c + 1, nbuf)

        p0 = wid * PPW + c * CP
        pltpu.sync_copy(pos_hbm.at[pl.ds(p0, CP)], posbuf)

        def prep(i, _):
            for k in range(KV):
                kk = pl.ds(k * LANES, LANES)
                posbuf[i, kk] = posbuf[i, kk] + tbuf[0, kk]
            return 0
        lax.fori_loop(0, CP, prep, 0)

        gathers[buf].wait()
        wbuf = wordbufs[buf]

        def tok_body(t, _):
            i = t & (LANES - 1)
            base = t - i
            tiv = tidbuf[buf, pl.ds(base, LANES)].astype(jnp.float32)
            tsplat = tiv.at[jnp.full((LANES,), i, jnp.int32)].get(
                mode="promise_in_bounds")
            xs = []
            s = jnp.zeros((LANES,), jnp.float32)
            q = jnp.zeros((LANES,), jnp.float32)
            for k in range(KV):
                kk = pl.ds(k * LANES, LANES)
                v = wbuf[t, kk] + posbuf[i, kk] + tsplat * dbuf[kk]
                xs.append(v)
                s = s + v
                q = q + v * v
            mean = jnp.sum(s) * (1.0 / H)
            var = jnp.maximum(jnp.sum(q) * (1.0 / H) - mean * mean, 0.0)
            inv = _rsqrt(var + EPS)
            shift = mean * inv
            for k in range(KV):
                kk = pl.ds(k * LANES, LANES)
                wbuf[t, kk] = (xs[k] * inv - shift) * gbuf[kk] + bbuf[kk]
            return 0
        lax.fori_loop(0, TPC, tok_body, 0)

        out_dmas[buf] = [
            pltpu.async_copy(wbuf.at[pl.ds(b * CP, CP)],
                             out_hbm.at[b, pl.ds(p0, CP)], semos[buf])
            for b in range(B)]

    for slot in range(2):
        if out_dmas[slot] is not None:
            for dma in out_dmas[slot]:
                dma.wait()


def kernel(input_ids, token_type_ids, word_emb, pos_emb, type_emb, gamma, beta):
    mesh = plsc.VectorSubcoreMesh(core_axis_name="c", subcore_axis_name="s",
                                  num_cores=NC, num_subcores=NS)
    k = pl.kernel(
        _body,
        out_type=jax.ShapeDtypeStruct((B, S, H), jnp.float32),
        mesh=mesh,
        compiler_params=pltpu.CompilerParams(needs_layout_passes=False),
        scratch_types=[
            pltpu.VMEM((2, TPC), jnp.int32),
            pltpu.VMEM((2, TPC), jnp.int32),
            pltpu.VMEM((TPC, H), jnp.float32),
            pltpu.VMEM((TPC, H), jnp.float32),
            pltpu.VMEM((CP, H), jnp.float32),
            pltpu.VMEM((2, H), jnp.float32),
            pltpu.VMEM((H,), jnp.float32),
            pltpu.VMEM((H,), jnp.float32),
            pltpu.VMEM((H,), jnp.float32),
            pltpu.SemaphoreType.DMA,
            pltpu.SemaphoreType.DMA,
            pltpu.SemaphoreType.DMA,
            pltpu.SemaphoreType.DMA,
        ],
    )
    return k(input_ids, token_type_ids, word_emb, pos_emb, type_emb,
             gamma, beta)


# speedup vs baseline: 3.9950x; 2.5756x over previous
import jax
import jax.numpy as jnp
from jax import lax
from jax.experimental import pallas as pl
from jax.experimental.pallas import tpu as pltpu
from jax.experimental.pallas import tpu_sc as plsc

B, S, H = 4, 2048, 768
NC, NS = 2, 16
NW = NC * NS
PPW = S // NW
EPS = 1e-12

BLK = 512
NBLK = (B * S) // BLK
SBLK = S // BLK


def _sc_gather_body(ids_hbm, word_hbm, out_hbm,
                    idsbuf, buf0, buf1, semg0, semg1, semo0, semo1):
    wid = lax.axis_index("s") * NC + lax.axis_index("c")
    p0 = wid * PPW
    bufs = (buf0, buf1)
    semgs = (semg0, semg1)
    semos = (semo0, semo1)

    def stage(b, slot):
        pltpu.sync_copy(ids_hbm.at[b, pl.ds(p0, PPW)], idsbuf.at[slot])

    def gather(slot):
        return pltpu.async_copy(word_hbm.at[idsbuf.at[slot]],
                                bufs[slot], semgs[slot])

    stage(0, 0)
    gathers = [gather(0), None]
    outs = [None, None]
    for b in range(B):
        buf = b & 1
        nbuf = 1 - buf
        if b + 1 < B:
            stage(b + 1, nbuf)
            if outs[nbuf] is not None:
                outs[nbuf].wait()
            gathers[nbuf] = gather(nbuf)
        gathers[buf].wait()
        outs[buf] = pltpu.async_copy(
            bufs[buf], out_hbm.at[b, pl.ds(p0, PPW)], semos[buf])
    for slot in range(2):
        if outs[slot] is not None:
            outs[slot].wait()


def _tc_ln_body(g_ref, pos_ref, tid_ref, t_ref, gam_ref, bet_ref, o_ref):
    x = g_ref[...]
    tid = tid_ref[...]
    x = x + pos_ref[...] + jnp.where(tid == 0, t_ref[0:1, :], t_ref[1:2, :])
    mean = jnp.mean(x, axis=-1, keepdims=True)
    xc = x - mean
    var = jnp.mean(xc * xc, axis=-1, keepdims=True)
    o_ref[...] = xc * lax.rsqrt(var + EPS) * gam_ref[...] + bet_ref[...]


def kernel(input_ids, token_type_ids, word_emb, pos_emb, type_emb, gamma, beta):
    mesh = plsc.VectorSubcoreMesh(core_axis_name="c", subcore_axis_name="s",
                                  num_cores=NC, num_subcores=NS)
    sc_gather = pl.kernel(
        _sc_gather_body,
        out_type=jax.ShapeDtypeStruct((B, S, H), jnp.float32),
        mesh=mesh,
        compiler_params=pltpu.CompilerParams(needs_layout_passes=False),
        scratch_types=[
            pltpu.VMEM((2, PPW), jnp.int32),
            pltpu.VMEM((PPW, H), jnp.float32),
            pltpu.VMEM((PPW, H), jnp.float32),
            pltpu.SemaphoreType.DMA,
            pltpu.SemaphoreType.DMA,
            pltpu.SemaphoreType.DMA,
            pltpu.SemaphoreType.DMA,
        ],
    )
    gathered = sc_gather(input_ids, word_emb)

    ln = pl.pallas_call(
        _tc_ln_body,
        grid=(NBLK,),
        in_specs=[
            pl.BlockSpec((BLK, H), lambda i: (i, 0)),
            pl.BlockSpec((BLK, H), lambda i: (i % SBLK, 0)),
            pl.BlockSpec((BLK, 1), lambda i: (i, 0)),
            pl.BlockSpec((2, H), lambda i: (0, 0)),
            pl.BlockSpec((1, H), lambda i: (0, 0)),
            pl.BlockSpec((1, H), lambda i: (0, 0)),
        ],
        out_specs=pl.BlockSpec((BLK, H), lambda i: (i, 0)),
        out_shape=jax.ShapeDtypeStruct((B * S, H), jnp.float32),
    )
    out = ln(gathered.reshape(B * S, H), pos_emb,
             token_type_ids.reshape(B * S, 1), type_emb,
             gamma.reshape(1, H), beta.reshape(1, H))
    return out.reshape(B, S, H)


# speedup vs baseline: 4.3814x; 1.0967x over previous
import jax
import jax.numpy as jnp
from jax import lax
from jax.experimental import pallas as pl
from jax.experimental.pallas import tpu as pltpu
from jax.experimental.pallas import tpu_sc as plsc

B, S, H = 4, 2048, 768
NC, NS = 2, 16
NW = NC * NS
PPW = S // NW
EPS = 1e-12

BLK = 1024
SBLK = S // BLK


def _sc_gather_body(ids_hbm, word_hbm, out_hbm,
                    idsbuf, buf0, buf1, semg0, semg1, semo0, semo1):
    wid = lax.axis_index("s") * NC + lax.axis_index("c")
    p0 = wid * PPW
    bufs = (buf0, buf1)
    semgs = (semg0, semg1)
    semos = (semo0, semo1)

    def stage(b, slot):
        pltpu.sync_copy(ids_hbm.at[b, pl.ds(p0, PPW)], idsbuf.at[slot])

    def gather(slot):
        return pltpu.async_copy(word_hbm.at[idsbuf.at[slot]],
                                bufs[slot], semgs[slot])

    stage(0, 0)
    gathers = [gather(0), None]
    outs = [None, None]
    for b in range(B):
        buf = b & 1
        nbuf = 1 - buf
        if b + 1 < B:
            stage(b + 1, nbuf)
            if outs[nbuf] is not None:
                outs[nbuf].wait()
            gathers[nbuf] = gather(nbuf)
        gathers[buf].wait()
        outs[buf] = pltpu.async_copy(
            bufs[buf], out_hbm.at[b, pl.ds(p0, PPW)], semos[buf])
    for slot in range(2):
        if outs[slot] is not None:
            outs[slot].wait()


def _tc_ln_body(g_ref, pos_ref, tid_ref, t_ref, gam_ref, bet_ref, o_ref):
    x = g_ref[...]
    tid = tid_ref[...]
    x = x + pos_ref[...] + jnp.where(tid == 0, t_ref[0:1, :], t_ref[1:2, :])
    mean = jnp.mean(x, axis=-1, keepdims=True)
    xc = x - mean
    var = jnp.mean(xc * xc, axis=-1, keepdims=True)
    o_ref[...] = xc * lax.rsqrt(var + EPS) * gam_ref[...] + bet_ref[...]


def kernel(input_ids, token_type_ids, word_emb, pos_emb, type_emb, gamma, beta):
    mesh = plsc.VectorSubcoreMesh(core_axis_name="c", subcore_axis_name="s",
                                  num_cores=NC, num_subcores=NS)
    sc_gather = pl.kernel(
        _sc_gather_body,
        out_type=jax.ShapeDtypeStruct((B, S, H), jnp.float32),
        mesh=mesh,
        compiler_params=pltpu.CompilerParams(needs_layout_passes=False),
        scratch_types=[
            pltpu.VMEM((2, PPW), jnp.int32),
            pltpu.VMEM((PPW, H), jnp.float32),
            pltpu.VMEM((PPW, H), jnp.float32),
            pltpu.SemaphoreType.DMA,
            pltpu.SemaphoreType.DMA,
            pltpu.SemaphoreType.DMA,
            pltpu.SemaphoreType.DMA,
        ],
    )
    gathered = sc_gather(input_ids, word_emb)

    ln = pl.pallas_call(
        _tc_ln_body,
        grid=(SBLK, B),
        in_specs=[
            pl.BlockSpec((BLK, H), lambda i, j: (j * SBLK + i, 0)),
            pl.BlockSpec((BLK, H), lambda i, j: (i, 0)),
            pl.BlockSpec((BLK, 1), lambda i, j: (j * SBLK + i, 0)),
            pl.BlockSpec((2, H), lambda i, j: (0, 0)),
            pl.BlockSpec((1, H), lambda i, j: (0, 0)),
            pl.BlockSpec((1, H), lambda i, j: (0, 0)),
        ],
        out_specs=pl.BlockSpec((BLK, H), lambda i, j: (j * SBLK + i, 0)),
        out_shape=jax.ShapeDtypeStruct((B * S, H), jnp.float32),
    )
    out = ln(gathered.reshape(B * S, H), pos_emb,
             token_type_ids.reshape(B * S, 1), type_emb,
             gamma.reshape(1, H), beta.reshape(1, H))
    return out.reshape(B, S, H)
